# baseline (speedup 1.0000x reference)
import functools

import jax
import jax.numpy as jnp
from jax import lax
from jax.experimental import pallas as pl
from jax.experimental.pallas import tpu as pltpu
from jax.experimental.pallas import tpu_sc as plsc

N_USER = 10000
N_ITEM = 10000
E = 320000
D = 128

NC = 2
NS = 16
CH = 64
CPT = 320
BCH = 32
NB = 4
BPT = CPT // BCH
EPAD = NS * CPT * CH
ACC = 10240
RPT = ACC // NS


def _tile_relation(sid, x_hbm, src_hbm, dst_hbm, sum_out, cnt_out,
                   acc2, acc1, srcv, dstv, rows, ones, zrow,
                   sems_g, sem_s, sem_o, with_counts):
    def _zero_row(r, carry):
        for k in range(D // 16):
            rows[0, r, pl.ds(k * 16, 16)] = jnp.zeros((16,), jnp.float32)
        return carry
    lax.fori_loop(0, CH, _zero_row, 0)
    if with_counts:
        for k in range(CH // 16):
            ones[pl.ds(k * 16, 16)] = jnp.ones((16,), jnp.float32)
        def _zero_zrow(r, carry):
            zrow[pl.ds(r * 16, 16)] = jnp.zeros((16,), jnp.float32)
            return carry
        lax.fori_loop(0, RPT // 16, _zero_zrow, 0)

    for k in range(RPT // CH):
        pltpu.sync_copy(rows.at[0], acc2.at[pl.ds(sid * RPT + k * CH, CH)])
    assert RPT % CH == 0
    if with_counts:
        pltpu.sync_copy(zrow, acc1.at[pl.ds(sid * RPT, RPT)])
    plsc.subcore_barrier()

    def _block(b, carry):
        @pl.when(b == 0)
        def _():
            off = sid * CPT
            pltpu.sync_copy(src_hbm.at[pl.ds(off, BCH)], srcv)
            pltpu.sync_copy(dst_hbm.at[pl.ds(off, BCH)], dstv)
            for k in range(NB):
                pltpu.async_copy(x_hbm.at[srcv.at[k]], rows.at[k], sems_g[k])

        def _round(g, carry2):
            obs = []
            for k in range(NB):
                j = NB * g + k
                pltpu.make_async_copy(x_hbm.at[srcv.at[j]], rows.at[k],
                                      sems_g[k]).wait()
                s = pltpu.async_copy(rows.at[k], acc2.at[dstv.at[j]], sem_s,
                                     add=True)
                if with_counts:
                    obs.append(pltpu.async_copy(ones, acc1.at[dstv.at[j]],
                                                sem_o, add=True))
                s.wait()
                @pl.when(j + NB < BCH)
                def _():
                    pltpu.async_copy(x_hbm.at[srcv.at[j + NB]], rows.at[k],
                                     sems_g[k])
            for o in obs:
                o.wait()
            return carry2
        lax.fori_loop(0, BCH // NB, _round, 0)

        @pl.when(b + 1 < BPT)
        def _():
            nof = sid * CPT + (b + 1) * BCH
            pltpu.sync_copy(src_hbm.at[pl.ds(nof, BCH)], srcv)
            pltpu.sync_copy(dst_hbm.at[pl.ds(nof, BCH)], dstv)
            for k in range(NB):
                pltpu.async_copy(x_hbm.at[srcv.at[k]], rows.at[k], sems_g[k])
        return carry
    lax.fori_loop(0, BPT, _block, 0)

    plsc.subcore_barrier()
    for k in range(RPT // CH):
        off = sid * RPT + k * CH
        pltpu.sync_copy(acc2.at[pl.ds(off, CH)], sum_out.at[pl.ds(off, CH)])
    if with_counts:
        pltpu.sync_copy(acc1.at[pl.ds(sid * RPT, RPT)],
                        cnt_out.at[pl.ds(sid * RPT, RPT)])


@functools.cache
def _sc_aggregate_fn(with_counts):
    @functools.partial(
        pl.kernel,
        out_type=(
            jax.ShapeDtypeStruct((ACC, D), jnp.float32),
            jax.ShapeDtypeStruct((ACC,), jnp.float32),
            jax.ShapeDtypeStruct((ACC, D), jnp.float32),
            jax.ShapeDtypeStruct((ACC,), jnp.float32),
        ),
        mesh=plsc.VectorSubcoreMesh(core_axis_name="c", subcore_axis_name="s",
                                    num_cores=NC, num_subcores=NS),
        scratch_types=[
            pltpu.VMEM((BCH, CH), jnp.int32),
            pltpu.VMEM((BCH, CH), jnp.int32),
            pltpu.VMEM((NB, CH, D), jnp.float32),
            pltpu.VMEM((CH,), jnp.float32),
            pltpu.VMEM((RPT,), jnp.float32),
            pltpu.VMEM_SHARED((ACC, D), jnp.float32),
            pltpu.VMEM_SHARED((ACC,), jnp.float32),
        ] + [pltpu.SemaphoreType.DMA] * (NB + 2),
    )
    def _sc_aggregate(x_user_hbm, x_item_hbm, src_u2i, dst_u2i, src_i2u, dst_i2u,
                      sum_item, cnt_item, sum_user, cnt_user,
                      srcv, dstv, rows, ones, zrow, acc2, acc1, *sems):
        cid = lax.axis_index("c")
        sid = lax.axis_index("s")
        sems_g = sems[:NB]
        sem_s = sems[NB]
        sem_o = sems[NB + 1]

        @pl.when(cid == 0)
        def _():
            _tile_relation(sid, x_user_hbm, src_u2i, dst_u2i, sum_item, cnt_item,
                           acc2, acc1, srcv, dstv, rows, ones, zrow,
                           sems_g, sem_s, sem_o, with_counts)

        @pl.when(cid == 1)
        def _():
            _tile_relation(sid, x_item_hbm, src_i2u, dst_i2u, sum_user, cnt_user,
                           acc2, acc1, srcv, dstv, rows, ones, zrow,
                           sems_g, sem_s, sem_o, with_counts)

    return _sc_aggregate


def _dense_body(x_ref, sum_ref, cnt_ref, Wd_ref, bd_ref, Ws_ref, bs_ref,
                Wt_ref, Wb_ref, bu_ref, g_ref, bb_ref, o_ref):
    aggr = sum_ref[...] / jnp.maximum(cnt_ref[...], 1.0)
    hd = jnp.dot(x_ref[...], Wd_ref[...], preferred_element_type=jnp.float32) + bd_ref[...]
    hs = jnp.dot(aggr, Ws_ref[...], preferred_element_type=jnp.float32) + bs_ref[...]
    h = (jnp.dot(hd, Wt_ref[...], preferred_element_type=jnp.float32)
         + jnp.dot(hs, Wb_ref[...], preferred_element_type=jnp.float32)
         + bu_ref[...])
    n = jnp.float32(h.shape[0])
    m = jnp.sum(h, axis=0, keepdims=True) / n
    d = h - m
    v = jnp.sum(d * d, axis=0, keepdims=True) / n
    o = d * lax.rsqrt(v + 1e-5) * g_ref[...] + bb_ref[...]
    o_ref[...] = jnp.where(o >= 0, o, 0.01 * o)


def _dense(x, summ, cnt, Wd, bd, Ws, bs, Wu, bu, g, bb, n_dst):
    full = lambda shape: pl.BlockSpec(shape, lambda i: (0,) * len(shape))
    return pl.pallas_call(
        _dense_body,
        grid=(1,),
        out_shape=jax.ShapeDtypeStruct((n_dst, D), jnp.float32),
        in_specs=[
            full((n_dst, D)), full((n_dst, D)), full((n_dst, 1)),
            full((D, D)), full((1, D)), full((D, D)), full((1, D)),
            full((D, D)), full((D, D)), full((1, D)),
            full((1, D)), full((1, D)),
        ],
        out_specs=full((n_dst, D)),
    )(x, summ, cnt,
      Wd, bd.reshape(1, D), Ws, bs.reshape(1, D),
      Wu[:D], Wu[D:], bu.reshape(1, D),
      g.reshape(1, D), bb.reshape(1, D))


def _pad_edges(ei):
    src = ei[0].astype(jnp.int32)
    dst = ei[1].astype(jnp.int32)
    pad = EPAD - E
    src_p = jnp.concatenate([src, jnp.zeros((pad,), jnp.int32)])
    dst_p = jnp.concatenate([dst, jnp.full((pad,), ACC - 1, jnp.int32)])
    return src_p.reshape(NS * CPT, CH), dst_p.reshape(NS * CPT, CH)


def kernel(x_user, x_item, edge_index_u2i, edge_index_i2u, l1_u2i_Wdst, l1_u2i_bdst, l1_u2i_Wsrc, l1_u2i_bsrc, l1_u2i_Wupd, l1_u2i_bupd, l1_i2u_Wdst, l1_i2u_bdst, l1_i2u_Wsrc, l1_i2u_bsrc, l1_i2u_Wupd, l1_i2u_bupd, l2_u2i_Wdst, l2_u2i_bdst, l2_u2i_Wsrc, l2_u2i_bsrc, l2_u2i_Wupd, l2_u2i_bupd, l2_i2u_Wdst, l2_i2u_bdst, l2_i2u_Wsrc, l2_i2u_bsrc, l2_i2u_Wupd, l2_i2u_bupd, bn1_user_g, bn1_user_b, bn1_item_g, bn1_item_b, bn2_user_g, bn2_user_b, bn2_item_g, bn2_item_b):
    src_u2i, dst_u2i = _pad_edges(edge_index_u2i)
    src_i2u, dst_i2u = _pad_edges(edge_index_i2u)

    sum_i, cnt_i, sum_u, cnt_u = _sc_aggregate_fn(True)(
        x_user, x_item, src_u2i, dst_u2i, src_i2u, dst_i2u)
    cnt_i2 = cnt_i.reshape(ACC, 1)
    cnt_u2 = cnt_u.reshape(ACC, 1)

    h_item = _dense(x_item, sum_i, cnt_i2,
                    l1_u2i_Wdst, l1_u2i_bdst, l1_u2i_Wsrc, l1_u2i_bsrc,
                    l1_u2i_Wupd, l1_u2i_bupd, bn1_item_g, bn1_item_b, N_ITEM)
    h_user = _dense(x_user, sum_u, cnt_u2,
                    l1_i2u_Wdst, l1_i2u_bdst, l1_i2u_Wsrc, l1_i2u_bsrc,
                    l1_i2u_Wupd, l1_i2u_bupd, bn1_user_g, bn1_user_b, N_USER)

    sum_i2, _, sum_u2, _ = _sc_aggregate_fn(False)(
        h_user, h_item, src_u2i, dst_u2i, src_i2u, dst_i2u)

    o_item = _dense(h_item, sum_i2, cnt_i2,
                    l2_u2i_Wdst, l2_u2i_bdst, l2_u2i_Wsrc, l2_u2i_bsrc,
                    l2_u2i_Wupd, l2_u2i_bupd, bn2_item_g, bn2_item_b, N_ITEM)
    o_user = _dense(h_user, sum_u2, cnt_u2,
                    l2_i2u_Wdst, l2_i2u_bdst, l2_i2u_Wsrc, l2_i2u_bsrc,
                    l2_i2u_Wupd, l2_i2u_bupd, bn2_user_g, bn2_user_b, N_USER)
    return (o_user, o_item)


# speedup vs baseline: 1.0679x; 1.0679x over previous
import functools

import jax
import jax.numpy as jnp
from jax import lax
from jax.experimental import pallas as pl
from jax.experimental.pallas import tpu as pltpu
from jax.experimental.pallas import tpu_sc as plsc

N_USER = 10000
N_ITEM = 10000
E = 320000
D = 128

NC = 2
NS = 16
CH = 128
CPT = 160
BCH = 32
NB = 2
BPT = CPT // BCH
EPAD = NS * CPT * CH
ACC = 10240
RPT = ACC // NS


def _tile_relation(sid, x_hbm, src_hbm, dst_hbm, sum_out, cnt_out,
                   acc2, acc1, srcv, dstv, rows, ones, zrow,
                   sems_g, sem_s, sem_o, with_counts):
    def _zero_row(r, carry):
        for k in range(D // 16):
            rows[0, r, pl.ds(k * 16, 16)] = jnp.zeros((16,), jnp.float32)
        return carry
    lax.fori_loop(0, CH, _zero_row, 0)
    if with_counts:
        for k in range(CH // 16):
            ones[pl.ds(k * 16, 16)] = jnp.ones((16,), jnp.float32)
        def _zero_zrow(r, carry):
            zrow[pl.ds(r * 16, 16)] = jnp.zeros((16,), jnp.float32)
            return carry
        lax.fori_loop(0, RPT // 16, _zero_zrow, 0)

    for k in range(RPT // CH):
        pltpu.sync_copy(rows.at[0], acc2.at[pl.ds(sid * RPT + k * CH, CH)])
    assert RPT % CH == 0
    if with_counts:
        pltpu.sync_copy(zrow, acc1.at[pl.ds(sid * RPT, RPT)])
    plsc.subcore_barrier()

    def _block(b, carry):
        @pl.when(b == 0)
        def _():
            off = sid * CPT
            pltpu.sync_copy(src_hbm.at[pl.ds(off, BCH)], srcv)
            pltpu.sync_copy(dst_hbm.at[pl.ds(off, BCH)], dstv)
            for k in range(NB):
                pltpu.async_copy(x_hbm.at[srcv.at[k]], rows.at[k], sems_g[k])

        def _round(g, carry2):
            obs = []
            for k in range(NB):
                j = NB * g + k
                pltpu.make_async_copy(x_hbm.at[srcv.at[j]], rows.at[k],
                                      sems_g[k]).wait()
                s = pltpu.async_copy(rows.at[k], acc2.at[dstv.at[j]], sem_s,
                                     add=True)
                if with_counts:
                    obs.append(pltpu.async_copy(ones, acc1.at[dstv.at[j]],
                                                sem_o, add=True))
                s.wait()
                @pl.when(j + NB < BCH)
                def _():
                    pltpu.async_copy(x_hbm.at[srcv.at[j + NB]], rows.at[k],
                                     sems_g[k])
            for o in obs:
                o.wait()
            return carry2
        lax.fori_loop(0, BCH // NB, _round, 0)

        @pl.when(b + 1 < BPT)
        def _():
            nof = sid * CPT + (b + 1) * BCH
            pltpu.sync_copy(src_hbm.at[pl.ds(nof, BCH)], srcv)
            pltpu.sync_copy(dst_hbm.at[pl.ds(nof, BCH)], dstv)
            for k in range(NB):
                pltpu.async_copy(x_hbm.at[srcv.at[k]], rows.at[k], sems_g[k])
        return carry
    lax.fori_loop(0, BPT, _block, 0)

    plsc.subcore_barrier()
    for k in range(RPT // CH):
        off = sid * RPT + k * CH
        pltpu.sync_copy(acc2.at[pl.ds(off, CH)], sum_out.at[pl.ds(off, CH)])
    if with_counts:
        pltpu.sync_copy(acc1.at[pl.ds(sid * RPT, RPT)],
                        cnt_out.at[pl.ds(sid * RPT, RPT)])


@functools.cache
def _sc_aggregate_fn(with_counts):
    @functools.partial(
        pl.kernel,
        out_type=(
            jax.ShapeDtypeStruct((ACC, D), jnp.float32),
            jax.ShapeDtypeStruct((ACC,), jnp.float32),
            jax.ShapeDtypeStruct((ACC, D), jnp.float32),
            jax.ShapeDtypeStruct((ACC,), jnp.float32),
        ),
        mesh=plsc.VectorSubcoreMesh(core_axis_name="c", subcore_axis_name="s",
                                    num_cores=NC, num_subcores=NS),
        scratch_types=[
            pltpu.VMEM((BCH, CH), jnp.int32),
            pltpu.VMEM((BCH, CH), jnp.int32),
            pltpu.VMEM((NB, CH, D), jnp.float32),
            pltpu.VMEM((CH,), jnp.float32),
            pltpu.VMEM((RPT,), jnp.float32),
            pltpu.VMEM_SHARED((ACC, D), jnp.float32),
            pltpu.VMEM_SHARED((ACC,), jnp.float32),
        ] + [pltpu.SemaphoreType.DMA] * (NB + 2),
    )
    def _sc_aggregate(x_user_hbm, x_item_hbm, src_u2i, dst_u2i, src_i2u, dst_i2u,
                      sum_item, cnt_item, sum_user, cnt_user,
                      srcv, dstv, rows, ones, zrow, acc2, acc1, *sems):
        cid = lax.axis_index("c")
        sid = lax.axis_index("s")
        sems_g = sems[:NB]
        sem_s = sems[NB]
        sem_o = sems[NB + 1]

        @pl.when(cid == 0)
        def _():
            _tile_relation(sid, x_user_hbm, src_u2i, dst_u2i, sum_item, cnt_item,
                           acc2, acc1, srcv, dstv, rows, ones, zrow,
                           sems_g, sem_s, sem_o, with_counts)

        @pl.when(cid == 1)
        def _():
            _tile_relation(sid, x_item_hbm, src_i2u, dst_i2u, sum_user, cnt_user,
                           acc2, acc1, srcv, dstv, rows, ones, zrow,
                           sems_g, sem_s, sem_o, with_counts)

    return _sc_aggregate


def _dense_body(x_ref, sum_ref, cnt_ref, Wd_ref, bd_ref, Ws_ref, bs_ref,
                Wt_ref, Wb_ref, bu_ref, g_ref, bb_ref, o_ref):
    aggr = sum_ref[...] / jnp.maximum(cnt_ref[...], 1.0)
    hd = jnp.dot(x_ref[...], Wd_ref[...], preferred_element_type=jnp.float32) + bd_ref[...]
    hs = jnp.dot(aggr, Ws_ref[...], preferred_element_type=jnp.float32) + bs_ref[...]
    h = (jnp.dot(hd, Wt_ref[...], preferred_element_type=jnp.float32)
         + jnp.dot(hs, Wb_ref[...], preferred_element_type=jnp.float32)
         + bu_ref[...])
    n = jnp.float32(h.shape[0])
    m = jnp.sum(h, axis=0, keepdims=True) / n
    d = h - m
    v = jnp.sum(d * d, axis=0, keepdims=True) / n
    o = d * lax.rsqrt(v + 1e-5) * g_ref[...] + bb_ref[...]
    o_ref[...] = jnp.where(o >= 0, o, 0.01 * o)


def _dense(x, summ, cnt, Wd, bd, Ws, bs, Wu, bu, g, bb, n_dst):
    full = lambda shape: pl.BlockSpec(shape, lambda i: (0,) * len(shape))
    return pl.pallas_call(
        _dense_body,
        grid=(1,),
        out_shape=jax.ShapeDtypeStruct((n_dst, D), jnp.float32),
        in_specs=[
            full((n_dst, D)), full((n_dst, D)), full((n_dst, 1)),
            full((D, D)), full((1, D)), full((D, D)), full((1, D)),
            full((D, D)), full((D, D)), full((1, D)),
            full((1, D)), full((1, D)),
        ],
        out_specs=full((n_dst, D)),
    )(x, summ, cnt,
      Wd, bd.reshape(1, D), Ws, bs.reshape(1, D),
      Wu[:D], Wu[D:], bu.reshape(1, D),
      g.reshape(1, D), bb.reshape(1, D))


def _pad_edges(ei):
    src = ei[0].astype(jnp.int32)
    dst = ei[1].astype(jnp.int32)
    pad = EPAD - E
    src_p = jnp.concatenate([src, jnp.zeros((pad,), jnp.int32)])
    dst_p = jnp.concatenate([dst, jnp.full((pad,), ACC - 1, jnp.int32)])
    return src_p.reshape(NS * CPT, CH), dst_p.reshape(NS * CPT, CH)


def kernel(x_user, x_item, edge_index_u2i, edge_index_i2u, l1_u2i_Wdst, l1_u2i_bdst, l1_u2i_Wsrc, l1_u2i_bsrc, l1_u2i_Wupd, l1_u2i_bupd, l1_i2u_Wdst, l1_i2u_bdst, l1_i2u_Wsrc, l1_i2u_bsrc, l1_i2u_Wupd, l1_i2u_bupd, l2_u2i_Wdst, l2_u2i_bdst, l2_u2i_Wsrc, l2_u2i_bsrc, l2_u2i_Wupd, l2_u2i_bupd, l2_i2u_Wdst, l2_i2u_bdst, l2_i2u_Wsrc, l2_i2u_bsrc, l2_i2u_Wupd, l2_i2u_bupd, bn1_user_g, bn1_user_b, bn1_item_g, bn1_item_b, bn2_user_g, bn2_user_b, bn2_item_g, bn2_item_b):
    src_u2i, dst_u2i = _pad_edges(edge_index_u2i)
    src_i2u, dst_i2u = _pad_edges(edge_index_i2u)

    sum_i, cnt_i, sum_u, cnt_u = _sc_aggregate_fn(True)(
        x_user, x_item, src_u2i, dst_u2i, src_i2u, dst_i2u)
    cnt_i2 = cnt_i.reshape(ACC, 1)
    cnt_u2 = cnt_u.reshape(ACC, 1)

    h_item = _dense(x_item, sum_i, cnt_i2,
                    l1_u2i_Wdst, l1_u2i_bdst, l1_u2i_Wsrc, l1_u2i_bsrc,
                    l1_u2i_Wupd, l1_u2i_bupd, bn1_item_g, bn1_item_b, N_ITEM)
    h_user = _dense(x_user, sum_u, cnt_u2,
                    l1_i2u_Wdst, l1_i2u_bdst, l1_i2u_Wsrc, l1_i2u_bsrc,
                    l1_i2u_Wupd, l1_i2u_bupd, bn1_user_g, bn1_user_b, N_USER)

    sum_i2, _, sum_u2, _ = _sc_aggregate_fn(False)(
        h_user, h_item, src_u2i, dst_u2i, src_i2u, dst_i2u)

    o_item = _dense(h_item, sum_i2, cnt_i2,
                    l2_u2i_Wdst, l2_u2i_bdst, l2_u2i_Wsrc, l2_u2i_bsrc,
                    l2_u2i_Wupd, l2_u2i_bupd, bn2_item_g, bn2_item_b, N_ITEM)
    o_user = _dense(h_user, sum_u2, cnt_u2,
                    l2_i2u_Wdst, l2_i2u_bdst, l2_i2u_Wsrc, l2_i2u_bsrc,
                    l2_i2u_Wupd, l2_i2u_bupd, bn2_user_g, bn2_user_b, N_USER)
    return (o_user, o_item)


# speedup vs baseline: 1.0796x; 1.0109x over previous
import functools

import jax
import jax.numpy as jnp
from jax import lax
from jax.experimental import pallas as pl
from jax.experimental.pallas import tpu as pltpu
from jax.experimental.pallas import tpu_sc as plsc

N_USER = 10000
N_ITEM = 10000
E = 320000
D = 128

NC = 2
NS = 16
CH = 128
CPT = 160
BCH = 32
NB = 2
BPT = CPT // BCH
EPAD = NS * CPT * CH
ACC = 10240
RPT = ACC // NS


def _tile_relation(sid, x_hbm, src_hbm, dst_hbm, sum_out, cnt_out,
                   acc2, acc1, srcv, dstv, rows, ones, zrow,
                   sems_g, sem_s, sem_o, with_counts):
    def _zero_row(r, carry):
        for k in range(D // 16):
            rows[0, r, pl.ds(k * 16, 16)] = jnp.zeros((16,), jnp.float32)
        return carry
    lax.fori_loop(0, CH, _zero_row, 0)
    if with_counts:
        for k in range(CH // 16):
            ones[pl.ds(k * 16, 16)] = jnp.ones((16,), jnp.float32)
        def _zero_zrow(r, carry):
            zrow[pl.ds(r * 16, 16)] = jnp.zeros((16,), jnp.float32)
            return carry
        lax.fori_loop(0, RPT // 16, _zero_zrow, 0)

    for k in range(RPT // CH):
        pltpu.sync_copy(rows.at[0], acc2.at[pl.ds(sid * RPT + k * CH, CH)])
    if with_counts:
        pltpu.sync_copy(zrow, acc1.at[pl.ds(sid * RPT, RPT)])
    plsc.subcore_barrier()

    def _block(b, carry):
        @pl.when(b == 0)
        def _():
            off = sid * CPT
            pltpu.sync_copy(src_hbm.at[pl.ds(off, BCH)], srcv)
            pltpu.sync_copy(dst_hbm.at[pl.ds(off, BCH)], dstv)
            for k in range(NB):
                pltpu.async_copy(x_hbm.at[srcv.at[k]], rows.at[k], sems_g[k])

        def _round(g, carry2):
            obs = []
            for k in range(NB):
                j = NB * g + k
                pltpu.make_async_copy(x_hbm.at[srcv.at[j]], rows.at[k],
                                      sems_g[k]).wait()
                s = pltpu.async_copy(rows.at[k], acc2.at[dstv.at[j]], sem_s,
                                     add=True)
                if with_counts:
                    obs.append(pltpu.async_copy(ones, acc1.at[dstv.at[j]],
                                                sem_o, add=True))
                s.wait()
                @pl.when(j + NB < BCH)
                def _():
                    pltpu.async_copy(x_hbm.at[srcv.at[j + NB]], rows.at[k],
                                     sems_g[k])
            for o in obs:
                o.wait()
            return carry2
        lax.fori_loop(0, BCH // NB, _round, 0)

        @pl.when(b + 1 < BPT)
        def _():
            nof = sid * CPT + (b + 1) * BCH
            pltpu.sync_copy(src_hbm.at[pl.ds(nof, BCH)], srcv)
            pltpu.sync_copy(dst_hbm.at[pl.ds(nof, BCH)], dstv)
            for k in range(NB):
                pltpu.async_copy(x_hbm.at[srcv.at[k]], rows.at[k], sems_g[k])
        return carry
    lax.fori_loop(0, BPT, _block, 0)

    plsc.subcore_barrier()
    for k in range(RPT // CH):
        off = sid * RPT + k * CH
        pltpu.sync_copy(acc2.at[pl.ds(off, CH)], sum_out.at[pl.ds(off, CH)])
    if with_counts:
        pltpu.sync_copy(acc1.at[pl.ds(sid * RPT, RPT)],
                        cnt_out.at[pl.ds(sid * RPT, RPT)])


@functools.cache
def _sc_aggregate_fn(with_counts):
    @functools.partial(
        pl.kernel,
        out_type=(
            jax.ShapeDtypeStruct((ACC, D), jnp.float32),
            jax.ShapeDtypeStruct((ACC,), jnp.float32),
            jax.ShapeDtypeStruct((ACC, D), jnp.float32),
            jax.ShapeDtypeStruct((ACC,), jnp.float32),
        ),
        mesh=plsc.VectorSubcoreMesh(core_axis_name="c", subcore_axis_name="s",
                                    num_cores=NC, num_subcores=NS),
        scratch_types=[
            pltpu.VMEM((BCH, CH), jnp.int32),
            pltpu.VMEM((BCH, CH), jnp.int32),
            pltpu.VMEM((NB, CH, D), jnp.float32),
            pltpu.VMEM((CH,), jnp.float32),
            pltpu.VMEM((RPT,), jnp.float32),
            pltpu.VMEM_SHARED((ACC, D), jnp.float32),
            pltpu.VMEM_SHARED((ACC,), jnp.float32),
        ] + [pltpu.SemaphoreType.DMA] * (NB + 2),
    )
    def _sc_aggregate(x_user_hbm, x_item_hbm, src_u2i, dst_u2i, src_i2u, dst_i2u,
                      sum_item, cnt_item, sum_user, cnt_user,
                      srcv, dstv, rows, ones, zrow, acc2, acc1, *sems):
        cid = lax.axis_index("c")
        sid = lax.axis_index("s")
        sems_g = sems[:NB]
        sem_s = sems[NB]
        sem_o = sems[NB + 1]

        @pl.when(cid == 0)
        def _():
            _tile_relation(sid, x_user_hbm, src_u2i, dst_u2i, sum_item, cnt_item,
                           acc2, acc1, srcv, dstv, rows, ones, zrow,
                           sems_g, sem_s, sem_o, with_counts)

        @pl.when(cid == 1)
        def _():
            _tile_relation(sid, x_item_hbm, src_i2u, dst_i2u, sum_user, cnt_user,
                           acc2, acc1, srcv, dstv, rows, ones, zrow,
                           sems_g, sem_s, sem_o, with_counts)

    return _sc_aggregate


def _dense_body(x_ref, sum_ref, cnt_ref, Wd_ref, bd_ref, Ws_ref, bs_ref,
                Wt_ref, Wb_ref, bu_ref, g_ref, bb_ref, o_ref):
    aggr = sum_ref[...] / jnp.maximum(cnt_ref[...], 1.0)
    hd = jnp.dot(x_ref[...], Wd_ref[...], preferred_element_type=jnp.float32) + bd_ref[...]
    hs = jnp.dot(aggr, Ws_ref[...], preferred_element_type=jnp.float32) + bs_ref[...]
    h = (jnp.dot(hd, Wt_ref[...], preferred_element_type=jnp.float32)
         + jnp.dot(hs, Wb_ref[...], preferred_element_type=jnp.float32)
         + bu_ref[...])
    n = jnp.float32(h.shape[0])
    m = jnp.sum(h, axis=0, keepdims=True) / n
    d = h - m
    v = jnp.sum(d * d, axis=0, keepdims=True) / n
    o = d * lax.rsqrt(v + 1e-5) * g_ref[...] + bb_ref[...]
    o_ref[...] = jnp.where(o >= 0, o, 0.01 * o)


def _dense(x, summ, cnt, Wd, bd, Ws, bs, Wu, bu, g, bb, n_dst):
    full = lambda shape: pl.BlockSpec(shape, lambda i: (0,) * len(shape))
    return pl.pallas_call(
        _dense_body,
        grid=(1,),
        out_shape=jax.ShapeDtypeStruct((n_dst, D), jnp.float32),
        in_specs=[
            full((n_dst, D)), full((n_dst, D)), full((n_dst, 1)),
            full((D, D)), full((1, D)), full((D, D)), full((1, D)),
            full((D, D)), full((D, D)), full((1, D)),
            full((1, D)), full((1, D)),
        ],
        out_specs=full((n_dst, D)),
    )(x, summ, cnt,
      Wd, bd.reshape(1, D), Ws, bs.reshape(1, D),
      Wu[:D], Wu[D:], bu.reshape(1, D),
      g.reshape(1, D), bb.reshape(1, D))


def _pad_edges(ei):
    src = ei[0].astype(jnp.int32)
    dst = ei[1].astype(jnp.int32)
    pad = EPAD - E
    src_p = jnp.concatenate([src, jnp.zeros((pad,), jnp.int32)])
    dst_p = jnp.concatenate([dst, jnp.full((pad,), ACC - 1, jnp.int32)])
    return src_p.reshape(NS * CPT, CH), dst_p.reshape(NS * CPT, CH)


def kernel(x_user, x_item, edge_index_u2i, edge_index_i2u, l1_u2i_Wdst, l1_u2i_bdst, l1_u2i_Wsrc, l1_u2i_bsrc, l1_u2i_Wupd, l1_u2i_bupd, l1_i2u_Wdst, l1_i2u_bdst, l1_i2u_Wsrc, l1_i2u_bsrc, l1_i2u_Wupd, l1_i2u_bupd, l2_u2i_Wdst, l2_u2i_bdst, l2_u2i_Wsrc, l2_u2i_bsrc, l2_u2i_Wupd, l2_u2i_bupd, l2_i2u_Wdst, l2_i2u_bdst, l2_i2u_Wsrc, l2_i2u_bsrc, l2_i2u_Wupd, l2_i2u_bupd, bn1_user_g, bn1_user_b, bn1_item_g, bn1_item_b, bn2_user_g, bn2_user_b, bn2_item_g, bn2_item_b):
    src_u2i, dst_u2i = _pad_edges(edge_index_u2i)
    src_i2u, dst_i2u = _pad_edges(edge_index_i2u)

    sum_i, cnt_i, sum_u, cnt_u = _sc_aggregate_fn(True)(
        x_user, x_item, src_u2i, dst_u2i, src_i2u, dst_i2u)
    cnt_i2 = cnt_i.reshape(ACC, 1)
    cnt_u2 = cnt_u.reshape(ACC, 1)

    h_item = _dense(x_item, sum_i, cnt_i2,
                    l1_u2i_Wdst, l1_u2i_bdst, l1_u2i_Wsrc, l1_u2i_bsrc,
                    l1_u2i_Wupd, l1_u2i_bupd, bn1_item_g, bn1_item_b, N_ITEM)
    h_user = _dense(x_user, sum_u, cnt_u2,
                    l1_i2u_Wdst, l1_i2u_bdst, l1_i2u_Wsrc, l1_i2u_bsrc,
                    l1_i2u_Wupd, l1_i2u_bupd, bn1_user_g, bn1_user_b, N_USER)

    sum_i2, _, sum_u2, _ = _sc_aggregate_fn(False)(
        h_user, h_item, src_u2i, dst_u2i, src_i2u, dst_i2u)

    o_item = _dense(h_item, sum_i2, cnt_i2,
                    l2_u2i_Wdst, l2_u2i_bdst, l2_u2i_Wsrc, l2_u2i_bsrc,
                    l2_u2i_Wupd, l2_u2i_bupd, bn2_item_g, bn2_item_b, N_ITEM)
    o_user = _dense(h_user, sum_u2, cnt_u2,
                    l2_i2u_Wdst, l2_i2u_bdst, l2_i2u_Wsrc, l2_i2u_bsrc,
                    l2_i2u_Wupd, l2_i2u_bupd, bn2_user_g, bn2_user_b, N_USER)
    return (o_user, o_item)
